# Initial kernel scaffold; baseline (speedup 1.0000x reference)
import jax
import jax.numpy as jnp
from jax.experimental import pallas as pl


def kernel(x, A_indices, A_values):
    raise NotImplementedError("write your pallas kernel here")


# speedup vs baseline: 4.6580x; 4.6580x over previous
import functools

import jax
import jax.numpy as jnp
from jax import lax
from jax.experimental import pallas as pl
from jax.experimental.pallas import tpu as pltpu
from jax.experimental.pallas import tpu_sc as plsc

M_ROWS = 46080
NK = 65536
NNZ = 1048576
GAMMA_N = 180
FH = 32

NUM_TILES = 16
NNZ_PER_TILE = NNZ // NUM_TILES
CHUNK = 128
CHUNKS_PER_TILE = NNZ_PER_TILE // CHUNK
ROWS_PER_TILE = M_ROWS // NUM_TILES
ZROWS = 180


def _sc_body(xs_h, cols2_h, rows_h, vals_h, out_h,
             colv, rowv, valv, gbuf, zbuf, acc, sem):
    c = lax.axis_index("c")
    s = lax.axis_index("s")

    def z_body(i, carry):
        zbuf[i, pl.ds(0, 16)] = jnp.zeros((16,), jnp.float32)
        zbuf[i, pl.ds(16, 16)] = jnp.zeros((16,), jnp.float32)
        return carry
    lax.fori_loop(0, ZROWS, z_body, 0)

    def zc_body(t, carry):
        pltpu.sync_copy(zbuf, acc.at[pl.ds(s * ROWS_PER_TILE + t * ZROWS, ZROWS)])
        return carry
    lax.fori_loop(0, ROWS_PER_TILE // ZROWS, zc_body, 0)
    plsc.subcore_barrier()

    base0 = s * NNZ_PER_TILE

    def chunk_body(ch, carry):
        base = base0 + ch * CHUNK
        pltpu.sync_copy(cols2_h.at[c, pl.ds(base, CHUNK)], colv)
        pltpu.sync_copy(rows_h.at[pl.ds(base, CHUNK)], rowv)
        pltpu.sync_copy(vals_h.at[pl.ds(base, CHUNK)], valv)
        pltpu.async_copy(xs_h.at[colv], gbuf, sem).wait()

        def mul_body(i, c2):
            vv = valv[pl.ds(i * 16, 16)]
            for l in range(16):
                j = i * 16 + l
                v = vv[l]
                g0 = gbuf[j, pl.ds(0, 16)]
                gbuf[j, pl.ds(0, 16)] = g0 * v
                g1 = gbuf[j, pl.ds(16, 16)]
                gbuf[j, pl.ds(16, 16)] = g1 * v
            return c2
        lax.fori_loop(0, CHUNK // 16, mul_body, 0)

        pltpu.sync_copy(gbuf, acc.at[rowv], add=True)
        return carry
    lax.fori_loop(0, CHUNKS_PER_TILE, chunk_body, 0)

    plsc.subcore_barrier()

    r0 = s * ROWS_PER_TILE
    pltpu.sync_copy(acc.at[pl.ds(r0, ROWS_PER_TILE)],
                    out_h.at[c, pl.ds(r0, ROWS_PER_TILE)])


@jax.jit
def _spmm_sc(xs, cols2, rows, vals):
    mesh = plsc.VectorSubcoreMesh(core_axis_name="c", subcore_axis_name="s")
    f = pl.kernel(
        _sc_body,
        out_type=jax.ShapeDtypeStruct((2, M_ROWS, FH), jnp.float32),
        mesh=mesh,
        scratch_types=[
            pltpu.VMEM((CHUNK,), jnp.int32),
            pltpu.VMEM((CHUNK,), jnp.int32),
            pltpu.VMEM((CHUNK,), jnp.float32),
            pltpu.VMEM((CHUNK, FH), jnp.float32),
            pltpu.VMEM((ZROWS, FH), jnp.float32),
            pltpu.VMEM_SHARED((M_ROWS, FH), jnp.float32),
            pltpu.SemaphoreType.DMA,
        ],
        compiler_params=pltpu.CompilerParams(use_tc_tiling_on_sc=False),
    )
    return f(xs, cols2, rows, vals)


def kernel(x, A_indices, A_values):
    Bs, Cs, Ns, Ks = x.shape
    x2D = x.reshape(Bs * Cs, Ks * Ns)
    xs = x2D.reshape(2, FH, NK).transpose(0, 2, 1).reshape(2 * NK, FH)
    cols = A_indices[1]
    cols2 = jnp.stack([cols, cols + NK])
    halves = _spmm_sc(xs, cols2, A_indices[0], A_values)
    proj2D = halves.transpose(1, 0, 2).reshape(M_ROWS, 2 * FH)
    return proj2D.reshape(Bs, Cs, GAMMA_N, -1)


# speedup vs baseline: 11.1675x; 2.3975x over previous
import jax
import jax.numpy as jnp
from jax import lax
from jax.experimental import pallas as pl
from jax.experimental.pallas import tpu as pltpu
from jax.experimental.pallas import tpu_sc as plsc

M_ROWS = 46080
NK = 65536
NNZ = 1048576
GAMMA_N = 180
FH = 32

NUM_TILES = 16
CHUNK = 128
SUBS_PER_BLOCK = 16
BLOCK = CHUNK * SUBS_PER_BLOCK
NNZ_PER_TILE = NNZ // NUM_TILES
BLOCKS_PER_TILE = NNZ_PER_TILE // BLOCK
NCHUNK_ROWS = NNZ // CHUNK
ROWS_PER_TILE = M_ROWS // NUM_TILES
ZROWS = 180


def _sc_body(xs_h, cols2_h, rows_h, vals_h, out_h,
             colv, rowv, valv, gbuf, zbuf, acc, semg):
    c = lax.axis_index("c")
    s = lax.axis_index("s")

    def z_body(i, carry):
        zbuf[i, pl.ds(0, 16)] = jnp.zeros((16,), jnp.float32)
        zbuf[i, pl.ds(16, 16)] = jnp.zeros((16,), jnp.float32)
        return carry
    lax.fori_loop(0, ZROWS, z_body, 0)

    def zc_body(t, carry):
        pltpu.sync_copy(zbuf, acc.at[pl.ds(s * ROWS_PER_TILE + t * ZROWS, ZROWS)])
        return carry
    lax.fori_loop(0, ROWS_PER_TILE // ZROWS, zc_body, 0)
    plsc.subcore_barrier()

    row0 = s * (NNZ_PER_TILE // CHUNK)

    def mul_sub(k):
        g = gbuf.at[k & 1]

        def mul_body(i, c2):
            vv = valv[k, pl.ds(i * 16, 16)]
            for l in range(16):
                v = vv[l]
                j = i * 16 + l
                g0 = g[j, pl.ds(0, 16)]
                g[j, pl.ds(0, 16)] = g0 * v
                g1 = g[j, pl.ds(16, 16)]
                g[j, pl.ds(16, 16)] = g1 * v
            return c2
        lax.fori_loop(0, CHUNK // 16, mul_body, 0)

    def block_body(blk, carry):
        base = row0 + blk * SUBS_PER_BLOCK
        pltpu.sync_copy(cols2_h.at[c, pl.ds(base, SUBS_PER_BLOCK)], colv)
        pltpu.sync_copy(rows_h.at[pl.ds(base, SUBS_PER_BLOCK)], rowv)
        pltpu.sync_copy(vals_h.at[pl.ds(base, SUBS_PER_BLOCK)], valv)

        descs = [None] * SUBS_PER_BLOCK
        descs[0] = pltpu.async_copy(xs_h.at[colv.at[0]], gbuf.at[0], semg.at[0])
        for k in range(SUBS_PER_BLOCK):
            if k + 1 < SUBS_PER_BLOCK:
                descs[k + 1] = pltpu.async_copy(
                    xs_h.at[colv.at[k + 1]], gbuf.at[(k + 1) & 1],
                    semg.at[(k + 1) & 1])
            descs[k].wait()
            mul_sub(k)
            pltpu.sync_copy(gbuf.at[k & 1], acc.at[rowv.at[k]], add=True)
        return carry
    lax.fori_loop(0, BLOCKS_PER_TILE, block_body, 0)

    plsc.subcore_barrier()

    r0 = s * ROWS_PER_TILE
    pltpu.sync_copy(acc.at[pl.ds(r0, ROWS_PER_TILE)],
                    out_h.at[c, pl.ds(r0, ROWS_PER_TILE)])


@jax.jit
def _spmm_sc(xs, cols2, rows, vals):
    mesh = plsc.VectorSubcoreMesh(core_axis_name="c", subcore_axis_name="s")
    f = pl.kernel(
        _sc_body,
        out_type=jax.ShapeDtypeStruct((2, M_ROWS, FH), jnp.float32),
        mesh=mesh,
        scratch_types=[
            pltpu.VMEM((SUBS_PER_BLOCK, CHUNK), jnp.int32),
            pltpu.VMEM((SUBS_PER_BLOCK, CHUNK), jnp.int32),
            pltpu.VMEM((SUBS_PER_BLOCK, CHUNK), jnp.float32),
            pltpu.VMEM((2, CHUNK, FH), jnp.float32),
            pltpu.VMEM((ZROWS, FH), jnp.float32),
            pltpu.VMEM_SHARED((M_ROWS, FH), jnp.float32),
            pltpu.SemaphoreType.DMA((2,)),
        ],
        compiler_params=pltpu.CompilerParams(use_tc_tiling_on_sc=False),
    )
    return f(xs, cols2, rows, vals)


def kernel(x, A_indices, A_values):
    Bs, Cs, Ns, Ks = x.shape
    x2D = x.reshape(Bs * Cs, Ks * Ns)
    xs = x2D.reshape(2, FH, NK).transpose(0, 2, 1).reshape(2 * NK, FH)
    cols = A_indices[1]
    cols2 = jnp.stack([cols, cols + NK]).reshape(2, NCHUNK_ROWS, CHUNK)
    rows2 = A_indices[0].reshape(NCHUNK_ROWS, CHUNK)
    vals2 = A_values.reshape(NCHUNK_ROWS, CHUNK)
    halves = _spmm_sc(xs, cols2, rows2, vals2)
    proj2D = halves.transpose(1, 0, 2).reshape(M_ROWS, 2 * FH)
    return proj2D.reshape(Bs, Cs, GAMMA_N, -1)


# speedup vs baseline: 12.9686x; 1.1613x over previous
import jax
import jax.numpy as jnp
from jax import lax
from jax.experimental import pallas as pl
from jax.experimental.pallas import tpu as pltpu
from jax.experimental.pallas import tpu_sc as plsc

M_ROWS = 46080
NK = 65536
NNZ = 1048576
GAMMA_N = 180
FH = 32

NUM_TILES = 16
CHUNK = 128
SUBS_PER_BLOCK = 16
BLOCK = CHUNK * SUBS_PER_BLOCK
NNZ_PER_TILE = NNZ // NUM_TILES
BLOCKS_PER_TILE = NNZ_PER_TILE // BLOCK
NCHUNK_ROWS = NNZ // CHUNK
ROWS_PER_TILE = M_ROWS // NUM_TILES
ZROWS = 180
NBUF = 4


def _sc_body(xs_h, cols2_h, rows_h, vals_h, out_h,
             colv, rowv, valv, gbuf, zbuf, acc, semg, sems):
    c = lax.axis_index("c")
    s = lax.axis_index("s")

    def z_body(i, carry):
        zbuf[i, pl.ds(0, 16)] = jnp.zeros((16,), jnp.float32)
        zbuf[i, pl.ds(16, 16)] = jnp.zeros((16,), jnp.float32)
        return carry
    lax.fori_loop(0, ZROWS, z_body, 0)

    def zc_body(t, carry):
        pltpu.sync_copy(zbuf, acc.at[pl.ds(s * ROWS_PER_TILE + t * ZROWS, ZROWS)])
        return carry
    lax.fori_loop(0, ROWS_PER_TILE // ZROWS, zc_body, 0)
    plsc.subcore_barrier()

    row0 = s * (NNZ_PER_TILE // CHUNK)

    def mul_sub(k):
        g = gbuf.at[k % NBUF]

        def mul_body(i, c2):
            vv = valv[k, pl.ds(i * 16, 16)]
            for l in range(16):
                v = vv[l]
                j = i * 16 + l
                g0 = g[j, pl.ds(0, 16)]
                g[j, pl.ds(0, 16)] = g0 * v
                g1 = g[j, pl.ds(16, 16)]
                g[j, pl.ds(16, 16)] = g1 * v
            return c2
        lax.fori_loop(0, CHUNK // 16, mul_body, 0, unroll=4)

    def block_body(blk, carry):
        base = row0 + blk * SUBS_PER_BLOCK
        pltpu.sync_copy(cols2_h.at[c, pl.ds(base, SUBS_PER_BLOCK)], colv)
        pltpu.sync_copy(rows_h.at[pl.ds(base, SUBS_PER_BLOCK)], rowv)
        pltpu.sync_copy(vals_h.at[pl.ds(base, SUBS_PER_BLOCK)], valv)

        gd = [None] * SUBS_PER_BLOCK
        sd = [None] * SUBS_PER_BLOCK
        for k in range(NBUF - 1):
            gd[k] = pltpu.async_copy(
                xs_h.at[colv.at[k]], gbuf.at[k % NBUF], semg.at[k % NBUF])
        for k in range(SUBS_PER_BLOCK):
            gd[k].wait()
            mul_sub(k)
            sd[k] = pltpu.async_copy(
                gbuf.at[k % NBUF], acc.at[rowv.at[k]], sems.at[k % NBUF],
                add=True)
            nk = k + NBUF - 1
            if nk < SUBS_PER_BLOCK:
                if nk - NBUF >= 0:
                    sd[nk - NBUF].wait()
                gd[nk] = pltpu.async_copy(
                    xs_h.at[colv.at[nk]], gbuf.at[nk % NBUF],
                    semg.at[nk % NBUF])
        for k in range(SUBS_PER_BLOCK - NBUF, SUBS_PER_BLOCK):
            sd[k].wait()
        return carry
    lax.fori_loop(0, BLOCKS_PER_TILE, block_body, 0)

    plsc.subcore_barrier()

    r0 = s * ROWS_PER_TILE
    pltpu.sync_copy(acc.at[pl.ds(r0, ROWS_PER_TILE)],
                    out_h.at[c, pl.ds(r0, ROWS_PER_TILE)])


@jax.jit
def _spmm_sc(xs, cols2, rows, vals):
    mesh = plsc.VectorSubcoreMesh(core_axis_name="c", subcore_axis_name="s")
    f = pl.kernel(
        _sc_body,
        out_type=jax.ShapeDtypeStruct((2, M_ROWS, FH), jnp.float32),
        mesh=mesh,
        scratch_types=[
            pltpu.VMEM((SUBS_PER_BLOCK, CHUNK), jnp.int32),
            pltpu.VMEM((SUBS_PER_BLOCK, CHUNK), jnp.int32),
            pltpu.VMEM((SUBS_PER_BLOCK, CHUNK), jnp.float32),
            pltpu.VMEM((NBUF, CHUNK, FH), jnp.float32),
            pltpu.VMEM((ZROWS, FH), jnp.float32),
            pltpu.VMEM_SHARED((M_ROWS, FH), jnp.float32),
            pltpu.SemaphoreType.DMA((NBUF,)),
            pltpu.SemaphoreType.DMA((NBUF,)),
        ],
        compiler_params=pltpu.CompilerParams(use_tc_tiling_on_sc=False),
    )
    return f(xs, cols2, rows, vals)


def kernel(x, A_indices, A_values):
    Bs, Cs, Ns, Ks = x.shape
    x2D = x.reshape(Bs * Cs, Ks * Ns)
    xs = x2D.reshape(2, FH, NK).transpose(0, 2, 1).reshape(2 * NK, FH)
    cols = A_indices[1]
    cols2 = jnp.stack([cols, cols + NK]).reshape(2, NCHUNK_ROWS, CHUNK)
    rows2 = A_indices[0].reshape(NCHUNK_ROWS, CHUNK)
    vals2 = A_values.reshape(NCHUNK_ROWS, CHUNK)
    halves = _spmm_sc(xs, cols2, rows2, vals2)
    proj2D = halves.transpose(1, 0, 2).reshape(M_ROWS, 2 * FH)
    return proj2D.reshape(Bs, Cs, GAMMA_N, -1)
